# baseline (speedup 1.0000x reference)
import jax
import jax.numpy as jnp
from jax.experimental import pallas as pl

NUM_CLASSES_ = 1000
ROWS_PER_BLOCK = 1024


def _onehot_block(x_ref, o_ref):
    ids = jax.lax.broadcasted_iota(jnp.int32, o_ref.shape, 1)
    o_ref[...] = (ids == x_ref[...]).astype(o_ref.dtype)


def kernel(x):
    out_dtype = jnp.zeros((), jnp.int64).dtype
    b, s = x.shape
    n = b * s
    x2 = x.reshape(n, 1).astype(jnp.int32)
    grid = n // ROWS_PER_BLOCK
    out = pl.pallas_call(
        _onehot_block,
        grid=(grid,),
        in_specs=[pl.BlockSpec((ROWS_PER_BLOCK, 1), lambda i: (i, 0))],
        out_specs=pl.BlockSpec((ROWS_PER_BLOCK, NUM_CLASSES_), lambda i: (i, 0)),
        out_shape=jax.ShapeDtypeStruct((n, NUM_CLASSES_), out_dtype),
    )(x2)
    return out.reshape(b, s, NUM_CLASSES_)


# speedup vs baseline: 1.4581x; 1.4581x over previous
import jax
import jax.numpy as jnp
from jax.experimental import pallas as pl

NUM_CLASSES_ = 1000
ROWS_PER_BLOCK = 64


def _onehot_block(x_ref, o_ref):
    ids = jax.lax.broadcasted_iota(jnp.int32, o_ref.shape, 2)
    xv = x_ref[...]
    o_ref[...] = (ids == xv[:, :, None]).astype(o_ref.dtype)


def kernel(x):
    out_dtype = jnp.zeros((), jnp.int64).dtype
    b, s = x.shape
    x = x.astype(jnp.int32)
    grid = b // ROWS_PER_BLOCK
    return pl.pallas_call(
        _onehot_block,
        grid=(grid,),
        in_specs=[pl.BlockSpec((ROWS_PER_BLOCK, s), lambda i: (i, 0))],
        out_specs=pl.BlockSpec(
            (ROWS_PER_BLOCK, s, NUM_CLASSES_), lambda i: (i, 0, 0)
        ),
        out_shape=jax.ShapeDtypeStruct((b, s, NUM_CLASSES_), out_dtype),
    )(x)
